# baseline (speedup 1.0000x reference)
import functools

import jax
import jax.numpy as jnp
from jax import lax
from jax.experimental import pallas as pl
from jax.experimental.pallas import tpu as pltpu
from jax.experimental.pallas import tpu_sc as plsc

N0 = 10000
N1 = 4096
N2 = 1024
E0 = 320000
E1 = 65536
D = 128
H = 128

NC = 2
NS = 16
NW = NC * NS
CH = 128


def _make_seg_sum(e_pad: int, n_stage: int, n_acc: int, nr: int, ch: int = CH):
    per_w = e_pad // NW
    nb = per_w // ch
    assert per_w % CH == 0 and e_pad % NW == 0 and n_acc % 16 == 0
    assert nb % nr == 0 and nb >= 2 * nr
    assert n_stage % NS == 0 and n_acc % NS == 0
    st_rows = n_stage // NS
    ac_rows = n_acc // NS

    mesh = plsc.VectorSubcoreMesh(core_axis_name="c", subcore_axis_name="s")

    @functools.partial(
        pl.kernel,
        out_type=(
            jax.ShapeDtypeStruct((NC, n_acc, D), jnp.float32),
            jax.ShapeDtypeStruct((NW, n_acc), jnp.float32),
        ),
        mesh=mesh,
        compiler_params=pltpu.CompilerParams(needs_layout_passes=False),
        scratch_types=[
            pltpu.VMEM((nb, ch), jnp.int32),
            pltpu.VMEM((nb, ch), jnp.int32),
            pltpu.VMEM((nr, ch, D), jnp.float32),
            pltpu.VMEM((n_acc,), jnp.float32),
            pltpu.VMEM_SHARED((n_stage, D), jnp.float32),
            pltpu.VMEM_SHARED((n_acc, D), jnp.float32),
            pltpu.SemaphoreType.DMA,
            pltpu.SemaphoreType.DMA,
            pltpu.SemaphoreType.DMA,
            pltpu.SemaphoreType.DMA,
            pltpu.SemaphoreType.DMA,
            pltpu.SemaphoreType.DMA,
            pltpu.SemaphoreType.DMA,
            pltpu.SemaphoreType.DMA,
        ],
    )
    def seg_kernel(x_hbm, src_hbm, dst_hbm, zagg_hbm, zhist_hbm,
                   agg_out, cnt_out,
                   srcb_v, dstb_v, rows_v, hist_v, x_sh, agg_sh,
                   *sems):
        c = lax.axis_index("c")
        s = lax.axis_index("s")
        wid = s * NC + c
        gsem = sems[:nr]
        ssem = sems[4:4 + nr]

        pltpu.sync_copy(zagg_hbm.at[pl.ds(s * ac_rows, ac_rows)],
                        agg_sh.at[pl.ds(s * ac_rows, ac_rows)])
        pltpu.sync_copy(x_hbm.at[pl.ds(s * st_rows, st_rows)],
                        x_sh.at[pl.ds(s * st_rows, st_rows)])
        pltpu.sync_copy(zhist_hbm, hist_v)
        row0 = wid * nb
        pltpu.sync_copy(src_hbm.at[pl.ds(row0, nb)], srcb_v)
        pltpu.sync_copy(dst_hbm.at[pl.ds(row0, nb)], dstb_v)
        plsc.subcore_barrier()

        ones = jnp.ones((16,), jnp.float32)

        for b in range(nr - 1):
            pltpu.async_copy(x_sh.at[srcb_v.at[b]], rows_v.at[b], gsem[b])

        def block(i, carry):
            for b in range(nr):
                k = nr * i + b
                pltpu.make_async_copy(
                    x_sh.at[srcb_v.at[k]], rows_v.at[b], gsem[b]).wait()
                pltpu.async_copy(
                    rows_v.at[b], agg_sh.at[dstb_v.at[k]], ssem[b], add=True)
                for j in range(ch // 16):
                    dv = dstb_v[k, pl.ds(j * 16, 16)]
                    plsc.addupdate_scatter(hist_v, [dv], ones)
                nxt = (b + nr - 1) % nr

                @pl.when(k + nr - 1 < nb)
                def _refill():
                    @pl.when(k > 0)
                    def _drain():
                        pltpu.make_async_copy(
                            rows_v.at[nxt],
                            agg_sh.at[dstb_v.at[k]],
                            ssem[nxt]).wait()
                    pltpu.async_copy(
                        x_sh.at[srcb_v.at[k + nr - 1]], rows_v.at[nxt],
                        gsem[nxt])
            return carry

        lax.fori_loop(0, nb // nr, block, 0)
        for b in range(nr):
            k = nb - nr + b
            pltpu.make_async_copy(
                rows_v.at[k % nr], agg_sh.at[dstb_v.at[k]],
                ssem[k % nr]).wait()

        pltpu.sync_copy(hist_v, cnt_out.at[wid])
        plsc.subcore_barrier()
        pltpu.sync_copy(agg_sh.at[pl.ds(s * ac_rows, ac_rows)],
                        agg_out.at[c, pl.ds(s * ac_rows, ac_rows)])

    return seg_kernel


def _combine(agg_parts, cnt_t, x_dst, w_l, b, w_r, leaky: bool):
    n = x_dst.shape[0]

    def body(ap, cp, xd, wl, bb, wr, o):
        agg = ap[0, :n, :] + ap[1, :n, :]
        cnt = jnp.sum(cp[...], axis=1, keepdims=True)
        mean = agg / jnp.maximum(cnt, 1.0)
        r = (lax.dot_general(mean, wl[...], (((1,), (1,)), ((), ())),
                             preferred_element_type=jnp.float32)
             + bb[...]
             + lax.dot_general(xd[...], wr[...], (((1,), (1,)), ((), ())),
                               preferred_element_type=jnp.float32))
        if leaky:
            r = jnp.where(r >= 0, r, 0.01 * r)
        o[...] = r

    return pl.pallas_call(
        body,
        out_shape=jax.ShapeDtypeStruct((n, H), jnp.float32),
    )(agg_parts, cnt_t, x_dst, w_l, b.reshape(1, H), w_r)


_BLK = NW * CH * 4
E0_PAD = ((E0 + _BLK - 1) // _BLK) * _BLK
E1_PAD = ((E1 + _BLK - 1) // _BLK) * _BLK
NACC0 = 33 * 128
NACC1 = 9 * 128

CH0 = CH
_seg0 = _make_seg_sum(E0_PAD, N1, NACC0, nr=2, ch=CH0)
_seg1 = _make_seg_sum(E1_PAD, N2, NACC1, nr=4)


def _pad_edges(ei, e_pad, n_dst, ch=CH):
    src, dst = ei[0], ei[1]
    pad = e_pad - src.shape[0]
    if pad:
        src = jnp.concatenate([src, jnp.zeros((pad,), src.dtype)])
        dst = jnp.concatenate([dst, jnp.full((pad,), n_dst, dst.dtype)])
    return src.reshape(e_pad // ch, ch), dst.reshape(e_pad // ch, ch)


def kernel(x, edge_index_0, edge_index_1, W1_l, b1, W1_r, W2_l, b2, W2_r):
    src0, dst0 = _pad_edges(edge_index_0, E0_PAD, N1, ch=CH0)
    src1, dst1 = _pad_edges(edge_index_1, E1_PAD, N2)

    zagg0 = jnp.zeros((NACC0, D), jnp.float32)
    zh0 = jnp.zeros((NACC0,), jnp.float32)
    zagg1 = jnp.zeros((NACC1, D), jnp.float32)
    zh1 = jnp.zeros((NACC1,), jnp.float32)

    agg0, cnt0 = _seg0(x, src0, dst0, zagg0, zh0)
    h = _combine(agg0, cnt0[:, :N1].T, x[:N1], W1_l, b1, W1_r, leaky=True)
    agg1, cnt1 = _seg1(h, src1, dst1, zagg1, zh1)
    out = _combine(agg1, cnt1[:, :N2].T, h[:N2], W2_l, b2, W2_r, leaky=False)
    return out
